# Initial kernel scaffold; baseline (speedup 1.0000x reference)
import jax
import jax.numpy as jnp
from jax.experimental import pallas as pl


def kernel(x, edge_index, edge_weight, W, b):
    raise NotImplementedError("write your pallas kernel here")


# speedup vs baseline: 4.5460x; 4.5460x over previous
import functools

import jax
import jax.numpy as jnp
from jax import lax
from jax.experimental import pallas as pl
from jax.experimental.pallas import tpu as pltpu
from jax.experimental.pallas import tpu_sc as plsc

NC = 2
NS = 16


@functools.lru_cache(maxsize=None)
def _sc_aggregate(N, D, E):
    NW = NC * NS
    e_per_w = E // NW
    K = 80
    n_chunks = e_per_w // K
    ZR = 208
    zrows = (N // (NS * 8)) * 8
    n_zero = zrows // ZR
    zrem = N - zrows * NS
    assert e_per_w * NW == E and n_chunks * K == e_per_w
    assert n_zero * ZR == zrows and zrem <= ZR
    assert D % 16 == 0

    mesh = plsc.VectorSubcoreMesh(core_axis_name="c", subcore_axis_name="s")

    @functools.partial(
        pl.kernel,
        out_type=jax.ShapeDtypeStruct((NC, N, D), jnp.float32),
        mesh=mesh,
        scratch_types=[
            pltpu.VMEM((K,), jnp.int32),
            pltpu.VMEM((K,), jnp.int32),
            pltpu.VMEM((K,), jnp.float32),
            pltpu.VMEM((K, D), jnp.float32),
            pltpu.VMEM((ZR, D), jnp.float32),
            pltpu.VMEM_SHARED((N, D), jnp.float32),
            pltpu.SemaphoreType.DMA,
        ],
    )
    def agg(x_hbm, src_hbm, dst_hbm, w_hbm, out_hbm,
            src_v, dst_v, w_v, rows_v, zero_v, acc_sh, sem):
        c = lax.axis_index("c")
        s = lax.axis_index("s")
        wid = c * NS + s

        zvec = jnp.zeros((16,), jnp.float32)

        def zrow(r, carry):
            for cb in range(D // 16):
                zero_v[r, pl.ds(cb * 16, 16)] = zvec
            return carry

        lax.fori_loop(0, ZR, zrow, 0)
        for z in range(n_zero):
            pltpu.sync_copy(zero_v, acc_sh.at[pl.ds(s * zrows + z * ZR, ZR)])
        if zrem:
            @pl.when(s == 0)
            def _():
                pltpu.sync_copy(zero_v.at[pl.ds(0, zrem)],
                                acc_sh.at[pl.ds(NS * zrows, zrem)])
        plsc.subcore_barrier()

        def chunk(i, carry):
            base = wid * e_per_w + i * K
            pltpu.sync_copy(src_hbm.at[pl.ds(base, K)], src_v)
            pltpu.sync_copy(dst_hbm.at[pl.ds(base, K)], dst_v)
            pltpu.sync_copy(w_hbm.at[pl.ds(base, K)], w_v)
            pltpu.async_copy(x_hbm.at[src_v], rows_v, sem).wait()

            def edge16(t, carry2):
                wv = w_v[pl.ds(t * 16, 16)]
                for l in range(16):
                    wj = wv[l]
                    j = t * 16 + l
                    for cb in range(D // 16):
                        rows_v[j, pl.ds(cb * 16, 16)] = (
                            rows_v[j, pl.ds(cb * 16, 16)] * wj
                        )
                return carry2

            lax.fori_loop(0, K // 16, edge16, 0)
            pltpu.sync_copy(rows_v, acc_sh.at[dst_v], add=True)
            return carry

        lax.fori_loop(0, n_chunks, chunk, 0)
        plsc.subcore_barrier()

        @pl.when(s == 0)
        def _():
            pltpu.sync_copy(acc_sh, out_hbm.at[c])

    return agg


@functools.lru_cache(maxsize=None)
def _tc_finish(N, D):
    BLK = 1000
    assert N % BLK == 0

    def body(p_ref, w_ref, b_ref, o_ref):
        acc = p_ref[0] + p_ref[1]
        o_ref[...] = (
            jnp.dot(acc, w_ref[...], preferred_element_type=jnp.float32)
            + b_ref[...]
        )

    return pl.pallas_call(
        body,
        grid=(N // BLK,),
        in_specs=[
            pl.BlockSpec((NC, BLK, D), lambda i: (0, i, 0)),
            pl.BlockSpec((D, D), lambda i: (0, 0)),
            pl.BlockSpec((1, D), lambda i: (0, 0)),
        ],
        out_specs=pl.BlockSpec((BLK, D), lambda i: (i, 0)),
        out_shape=jax.ShapeDtypeStruct((N, D), jnp.float32),
    )


def kernel(x, edge_index, edge_weight, W, b):
    N, D = x.shape
    E = edge_weight.shape[0]
    partials = _sc_aggregate(N, D, E)(
        x, edge_index[0], edge_index[1], edge_weight)
    return _tc_finish(N, D)(partials, W, b.reshape(1, D))
